# Initial kernel scaffold; baseline (speedup 1.0000x reference)
import jax
import jax.numpy as jnp
from jax.experimental import pallas as pl


def kernel(x, edge_index, W1_l, W1_r, b1, W2_l, W2_r, b2):
    raise NotImplementedError("write your pallas kernel here")


# speedup vs baseline: 8.1113x; 8.1113x over previous
import functools

import jax
import jax.numpy as jnp
from jax import lax
from jax.experimental import pallas as pl
from jax.experimental.pallas import tpu as pltpu
from jax.experimental.pallas import tpu_sc as plsc

N = 10000
E = 320000
D_IN = 128
D_HID = 1024
D_OUT = 128
DH = 64

NC = 2
NS = 16
L = 128

EPT = 20480
EP = EPT * NS
ROWS_PER_T = EPT // L
CB = 4
NCHUNK = ROWS_PER_T // CB

N_PAD = 10112
RPT = N_PAD // NS


def _seg_sum_body(with_count, *refs):
    if with_count:
        (tab_hbm, ei_hbm, zrow_hbm, zcnt_hbm, ones_hbm,
         out_hbm, cnt_hbm,
         srcv, dstv, rows, ones, acc, cacc, sem) = refs
    else:
        (tab_hbm, ei_hbm, zrow_hbm,
         out_hbm,
         srcv, dstv, rows, acc, sem) = refs

    cid = lax.axis_index("c")
    sid = lax.axis_index("s")

    pltpu.sync_copy(zrow_hbm, acc.at[pl.ds(sid * RPT, RPT)])
    if with_count:
        @pl.when(cid == 0)
        def _():
            pltpu.sync_copy(zcnt_hbm, cacc.at[pl.ds(sid * RPT, RPT)])
            pltpu.sync_copy(ones_hbm, ones)
    plsc.subcore_barrier()

    row0 = sid * ROWS_PER_T

    def chunk(c, carry):
        r = pl.multiple_of(row0 + c * CB, CB)
        pltpu.sync_copy(ei_hbm.at[0, pl.ds(r, CB)], srcv)
        pltpu.sync_copy(ei_hbm.at[1, pl.ds(r, CB)], dstv)
        descs = [
            pltpu.async_copy(tab_hbm.at[cid].at[srcv.at[j]],
                             rows.at[pl.ds(j * L, L)], sem)
            for j in range(CB)
        ]
        for d in descs:
            d.wait()
        for j in range(CB):
            pltpu.sync_copy(rows.at[pl.ds(j * L, L)],
                            acc.at[dstv.at[j]], add=True)
        if with_count:
            @pl.when(cid == 0)
            def _():
                for j in range(CB):
                    pltpu.sync_copy(ones, cacc.at[dstv.at[j]], add=True)
        return carry

    lax.fori_loop(0, NCHUNK, chunk, 0)
    plsc.subcore_barrier()

    o = pl.ds(sid * RPT, RPT)
    pltpu.sync_copy(acc.at[o], out_hbm.at[cid, o])
    if with_count:
        @pl.when(cid == 0)
        def _():
            pltpu.sync_copy(cacc.at[o], cnt_hbm.at[o])


def _make_seg_sum(with_count):
    out_type = [jax.ShapeDtypeStruct((NC, N_PAD, DH), jnp.float32)]
    scratch = [
        pltpu.VMEM((CB, L), jnp.int32),
        pltpu.VMEM((CB, L), jnp.int32),
        pltpu.VMEM((CB * L, DH), jnp.float32),
    ]
    if with_count:
        out_type.append(jax.ShapeDtypeStruct((N_PAD, 16), jnp.float32))
        scratch.append(pltpu.VMEM((L, 16), jnp.float32))
    scratch.append(pltpu.VMEM_SHARED((N_PAD, DH), jnp.float32))
    if with_count:
        scratch.append(pltpu.VMEM_SHARED((N_PAD, 16), jnp.float32))
    scratch.append(pltpu.SemaphoreType.DMA)

    return pl.kernel(
        functools.partial(_seg_sum_body, with_count),
        out_type=out_type,
        mesh=plsc.VectorSubcoreMesh(core_axis_name="c", subcore_axis_name="s"),
        scratch_types=scratch,
        compiler_params=pltpu.CompilerParams(use_tc_tiling_on_sc=False),
        name="seg_sum_cnt" if with_count else "seg_sum",
    )


_seg_sum_cnt = _make_seg_sum(True)
_seg_sum = _make_seg_sum(False)


BN = 1000


def _tc1_body(p_ref, c_ref, x_ref, wl_ref, wr_ref, b_ref, o_ref):
    mean = jnp.concatenate([p_ref[0], p_ref[1]], axis=1)
    mean = mean / jnp.maximum(c_ref[:, 0:1], 1.0)
    acc = jnp.dot(mean, wl_ref[...], preferred_element_type=jnp.float32)
    acc = acc + jnp.dot(x_ref[...], wr_ref[...],
                        preferred_element_type=jnp.float32)
    o_ref[...] = jnp.maximum(acc + b_ref[...], 0.0)


def _tc1(p1, c1, x, W1_l, W1_r, b1):
    return pl.pallas_call(
        _tc1_body,
        grid=(N // BN,),
        in_specs=[
            pl.BlockSpec((2, BN, DH), lambda i: (0, i, 0)),
            pl.BlockSpec((BN, 16), lambda i: (i, 0)),
            pl.BlockSpec((BN, D_IN), lambda i: (i, 0)),
            pl.BlockSpec((D_IN, D_HID), lambda i: (0, 0)),
            pl.BlockSpec((D_IN, D_HID), lambda i: (0, 0)),
            pl.BlockSpec((1, D_HID), lambda i: (0, 0)),
        ],
        out_specs=pl.BlockSpec((BN, D_HID), lambda i: (i, 0)),
        out_shape=jax.ShapeDtypeStruct((N, D_HID), jnp.float32),
    )(p1, c1, x, W1_l, W1_r, b1.reshape(1, D_HID))


def _tc2_body(h_ref, wl_ref, wr_ref, z_ref, w_ref):
    h = h_ref[...]
    z = jnp.dot(h, wl_ref[...], preferred_element_type=jnp.float32)
    z_ref[0] = z[:, :DH]
    z_ref[1] = z[:, DH:]
    w_ref[...] = jnp.dot(h, wr_ref[...], preferred_element_type=jnp.float32)


def _tc2(h, W2_l, W2_r):
    return pl.pallas_call(
        _tc2_body,
        grid=(N // BN,),
        in_specs=[
            pl.BlockSpec((BN, D_HID), lambda i: (i, 0)),
            pl.BlockSpec((D_HID, D_OUT), lambda i: (0, 0)),
            pl.BlockSpec((D_HID, D_OUT), lambda i: (0, 0)),
        ],
        out_specs=[
            pl.BlockSpec((2, BN, DH), lambda i: (0, i, 0)),
            pl.BlockSpec((BN, D_OUT), lambda i: (i, 0)),
        ],
        out_shape=[
            jax.ShapeDtypeStruct((2, N, DH), jnp.float32),
            jax.ShapeDtypeStruct((N, D_OUT), jnp.float32),
        ],
    )(h, W2_l, W2_r)


def _tc3_body(p_ref, c_ref, w_ref, b_ref, o_ref):
    mean = jnp.concatenate([p_ref[0], p_ref[1]], axis=1)
    o = mean / jnp.maximum(c_ref[:, 0:1], 1.0) + w_ref[...] + b_ref[...]
    m = jnp.max(o, axis=1, keepdims=True)
    lse = m + jnp.log(jnp.sum(jnp.exp(o - m), axis=1, keepdims=True))
    o_ref[...] = o - lse


def _tc3(p2, c1, w, b2):
    return pl.pallas_call(
        _tc3_body,
        grid=(N // BN,),
        in_specs=[
            pl.BlockSpec((2, BN, DH), lambda i: (0, i, 0)),
            pl.BlockSpec((BN, 16), lambda i: (i, 0)),
            pl.BlockSpec((BN, D_OUT), lambda i: (i, 0)),
            pl.BlockSpec((1, D_OUT), lambda i: (0, 0)),
        ],
        out_specs=pl.BlockSpec((BN, D_OUT), lambda i: (i, 0)),
        out_shape=jax.ShapeDtypeStruct((N, D_OUT), jnp.float32),
    )(p2, c1, w, b2.reshape(1, D_OUT))


def kernel(x, edge_index, W1_l, W1_r, b1, W2_l, W2_r, b2):
    ei = edge_index.astype(jnp.int32)
    pad_src = jnp.zeros((EP - E,), jnp.int32)
    pad_dst = jnp.full((EP - E,), N, jnp.int32)
    ei = jnp.concatenate([ei, jnp.stack([pad_src, pad_dst])], axis=1)
    ei = ei.reshape(2, EP // L, L)

    xt = x.reshape(N, NC, DH).transpose(1, 0, 2)

    zrow = jnp.zeros((RPT, DH), jnp.float32)
    zcnt = jnp.zeros((RPT, 16), jnp.float32)
    ones = jnp.ones((L, 16), jnp.float32)

    p1, c1 = _seg_sum_cnt(xt, ei, zrow, zcnt, ones)
    h = _tc1(p1, c1, x, W1_l, W1_r, b1)
    z, w = _tc2(h, W2_l, W2_r)
    (p2,) = _seg_sum(z, ei, zrow)
    return _tc3(p2, c1, w, b2)


# speedup vs baseline: 10.4338x; 1.2863x over previous
import functools

import jax
import jax.numpy as jnp
from jax import lax
from jax.experimental import pallas as pl
from jax.experimental.pallas import tpu as pltpu
from jax.experimental.pallas import tpu_sc as plsc

N = 10000
E = 320000
D_IN = 128
D_HID = 1024
D_OUT = 128
DH = 64

NC = 2
NS = 16
L = 128

EPT = 20480
EP = EPT * NS
ROWS_PER_T = EPT // L
CB = 2
NCHUNK = ROWS_PER_T // CB

N_PAD = 10112
RPT = N_PAD // NS


def _seg_sum_body(with_count, *refs):
    if with_count:
        (tab_hbm, ei_hbm, zrow_hbm, zcnt_hbm, ones_hbm,
         out_hbm, cnt_hbm,
         srcv, dstv, rows0, rows1, ones, acc, cacc,
         sem_g0, sem_g1, sem_s, sem_c) = refs
    else:
        (tab_hbm, ei_hbm, zrow_hbm,
         out_hbm,
         srcv, dstv, rows0, rows1, acc,
         sem_g0, sem_g1, sem_s) = refs

    cid = lax.axis_index("c")
    sid = lax.axis_index("s")

    pltpu.sync_copy(zrow_hbm, acc.at[pl.ds(sid * RPT, RPT)])
    if with_count:
        pltpu.sync_copy(zcnt_hbm, cacc.at[pl.ds(sid * RPT, RPT)])
        pltpu.sync_copy(ones_hbm, ones)
    row0 = sid * ROWS_PER_T
    pltpu.sync_copy(ei_hbm.at[0, pl.ds(row0, ROWS_PER_T)], srcv)
    pltpu.sync_copy(ei_hbm.at[1, pl.ds(row0, ROWS_PER_T)], dstv)
    plsc.subcore_barrier()

    def issue_gathers(buf, sem, r):
        for j in range(CB):
            pltpu.async_copy(tab_hbm.at[cid].at[srcv.at[r + j]],
                             buf.at[pl.ds(j * L, L)], sem)

    def drain_gathers(buf, sem):
        for j in range(CB):
            pltpu.make_async_copy(tab_hbm.at[cid].at[srcv.at[j]],
                                  buf.at[pl.ds(j * L, L)], sem).wait()

    def scatter_chunk(buf, r, count_core):
        for j in range(CB):
            pltpu.async_copy(buf.at[pl.ds(j * L, L)],
                             acc.at[dstv.at[r + j]], sem_s, add=True)
        if with_count:
            @pl.when(cid == count_core)
            def _():
                for j in range(CB):
                    pltpu.async_copy(ones, cacc.at[dstv.at[r + j]],
                                     sem_c, add=True)
                for j in range(CB):
                    pltpu.make_async_copy(ones, cacc.at[dstv.at[j]],
                                          sem_c).wait()
        for j in range(CB):
            pltpu.make_async_copy(buf.at[pl.ds(j * L, L)],
                                  acc.at[dstv.at[j]], sem_s).wait()

    issue_gathers(rows0, sem_g0, 0)
    issue_gathers(rows1, sem_g1, CB)

    def body(i, carry):
        r = 2 * i * CB
        drain_gathers(rows0, sem_g0)
        scatter_chunk(rows0, r, 0)

        @pl.when(r + 2 * CB < ROWS_PER_T)
        def _():
            issue_gathers(rows0, sem_g0, r + 2 * CB)

        drain_gathers(rows1, sem_g1)
        scatter_chunk(rows1, r + CB, 1)

        @pl.when(r + 3 * CB < ROWS_PER_T)
        def _():
            issue_gathers(rows1, sem_g1, r + 3 * CB)

        return carry

    lax.fori_loop(0, NCHUNK // 2, body, 0)
    plsc.subcore_barrier()

    o = pl.ds(sid * RPT, RPT)
    pltpu.sync_copy(acc.at[o], out_hbm.at[cid, o])
    if with_count:
        pltpu.sync_copy(cacc.at[o], cnt_hbm.at[cid, o])


def _make_seg_sum(with_count):
    out_type = [jax.ShapeDtypeStruct((NC, N_PAD, DH), jnp.float32)]
    scratch = [
        pltpu.VMEM((ROWS_PER_T, L), jnp.int32),
        pltpu.VMEM((ROWS_PER_T, L), jnp.int32),
        pltpu.VMEM((CB * L, DH), jnp.float32),
        pltpu.VMEM((CB * L, DH), jnp.float32),
    ]
    if with_count:
        out_type.append(jax.ShapeDtypeStruct((NC, N_PAD, 16), jnp.float32))
        scratch.append(pltpu.VMEM((L, 16), jnp.float32))
    scratch.append(pltpu.VMEM_SHARED((N_PAD, DH), jnp.float32))
    if with_count:
        scratch.append(pltpu.VMEM_SHARED((N_PAD, 16), jnp.float32))
    scratch.append(pltpu.SemaphoreType.DMA)
    scratch.append(pltpu.SemaphoreType.DMA)
    scratch.append(pltpu.SemaphoreType.DMA)
    if with_count:
        scratch.append(pltpu.SemaphoreType.DMA)

    return pl.kernel(
        functools.partial(_seg_sum_body, with_count),
        out_type=out_type,
        mesh=plsc.VectorSubcoreMesh(core_axis_name="c", subcore_axis_name="s"),
        scratch_types=scratch,
        compiler_params=pltpu.CompilerParams(use_tc_tiling_on_sc=False),
        name="seg_sum_cnt" if with_count else "seg_sum",
    )


_seg_sum_cnt = _make_seg_sum(True)
_seg_sum = _make_seg_sum(False)


BN = 1000


def _tc1_body(p_ref, c_ref, x_ref, wl_ref, wr_ref, b_ref, o_ref):
    mean = jnp.concatenate([p_ref[0], p_ref[1]], axis=1)
    mean = mean / jnp.maximum(c_ref[0, :, 0:1] + c_ref[1, :, 0:1], 1.0)
    acc = jnp.dot(mean, wl_ref[...], preferred_element_type=jnp.float32)
    acc = acc + jnp.dot(x_ref[...], wr_ref[...],
                        preferred_element_type=jnp.float32)
    o_ref[...] = jnp.maximum(acc + b_ref[...], 0.0)


def _tc1(p1, c1, x, W1_l, W1_r, b1):
    return pl.pallas_call(
        _tc1_body,
        grid=(N // BN,),
        in_specs=[
            pl.BlockSpec((2, BN, DH), lambda i: (0, i, 0)),
            pl.BlockSpec((2, BN, 16), lambda i: (0, i, 0)),
            pl.BlockSpec((BN, D_IN), lambda i: (i, 0)),
            pl.BlockSpec((D_IN, D_HID), lambda i: (0, 0)),
            pl.BlockSpec((D_IN, D_HID), lambda i: (0, 0)),
            pl.BlockSpec((1, D_HID), lambda i: (0, 0)),
        ],
        out_specs=pl.BlockSpec((BN, D_HID), lambda i: (i, 0)),
        out_shape=jax.ShapeDtypeStruct((N, D_HID), jnp.float32),
    )(p1, c1, x, W1_l, W1_r, b1.reshape(1, D_HID))


def _tc2_body(h_ref, wl_ref, wr_ref, z_ref, w_ref):
    h = h_ref[...]
    z = jnp.dot(h, wl_ref[...], preferred_element_type=jnp.float32)
    z_ref[0] = z[:, :DH]
    z_ref[1] = z[:, DH:]
    w_ref[...] = jnp.dot(h, wr_ref[...], preferred_element_type=jnp.float32)


def _tc2(h, W2_l, W2_r):
    return pl.pallas_call(
        _tc2_body,
        grid=(N // BN,),
        in_specs=[
            pl.BlockSpec((BN, D_HID), lambda i: (i, 0)),
            pl.BlockSpec((D_HID, D_OUT), lambda i: (0, 0)),
            pl.BlockSpec((D_HID, D_OUT), lambda i: (0, 0)),
        ],
        out_specs=[
            pl.BlockSpec((2, BN, DH), lambda i: (0, i, 0)),
            pl.BlockSpec((BN, D_OUT), lambda i: (i, 0)),
        ],
        out_shape=[
            jax.ShapeDtypeStruct((2, N, DH), jnp.float32),
            jax.ShapeDtypeStruct((N, D_OUT), jnp.float32),
        ],
    )(h, W2_l, W2_r)


def _tc3_body(p_ref, c_ref, w_ref, b_ref, o_ref):
    mean = jnp.concatenate([p_ref[0], p_ref[1]], axis=1)
    cnt = jnp.maximum(c_ref[0, :, 0:1] + c_ref[1, :, 0:1], 1.0)
    o = mean / cnt + w_ref[...] + b_ref[...]
    m = jnp.max(o, axis=1, keepdims=True)
    lse = m + jnp.log(jnp.sum(jnp.exp(o - m), axis=1, keepdims=True))
    o_ref[...] = o - lse


def _tc3(p2, c1, w, b2):
    return pl.pallas_call(
        _tc3_body,
        grid=(N // BN,),
        in_specs=[
            pl.BlockSpec((2, BN, DH), lambda i: (0, i, 0)),
            pl.BlockSpec((2, BN, 16), lambda i: (0, i, 0)),
            pl.BlockSpec((BN, D_OUT), lambda i: (i, 0)),
            pl.BlockSpec((1, D_OUT), lambda i: (0, 0)),
        ],
        out_specs=pl.BlockSpec((BN, D_OUT), lambda i: (i, 0)),
        out_shape=jax.ShapeDtypeStruct((N, D_OUT), jnp.float32),
    )(p2, c1, w, b2.reshape(1, D_OUT))


def kernel(x, edge_index, W1_l, W1_r, b1, W2_l, W2_r, b2):
    ei = edge_index.astype(jnp.int32)
    pad_src = jnp.zeros((EP - E,), jnp.int32)
    pad_dst = jnp.full((EP - E,), N, jnp.int32)
    ei = jnp.concatenate([ei, jnp.stack([pad_src, pad_dst])], axis=1)
    ei = ei.reshape(2, EP // L, L)

    xt = x.reshape(N, NC, DH).transpose(1, 0, 2)

    zrow = jnp.zeros((RPT, DH), jnp.float32)
    zcnt = jnp.zeros((RPT, 16), jnp.float32)
    ones = jnp.ones((L, 16), jnp.float32)

    p1, c1 = _seg_sum_cnt(xt, ei, zrow, zcnt, ones)
    h = _tc1(p1, c1, x, W1_l, W1_r, b1)
    z, w = _tc2(h, W2_l, W2_r)
    (p2,) = _seg_sum(z, ei, zrow)
    return _tc3(p2, c1, w, b2)
